# baseline (speedup 1.0000x reference)
import functools

import jax
import jax.numpy as jnp
from jax import lax
from jax.experimental import pallas as pl
from jax.experimental.pallas import tpu as pltpu
from jax.experimental.pallas import tpu_sc as plsc

_D = 1024
_MAX_LEN = 32768
_SEQ = 8192
_NC = 2
_NS = 16
_NW = _NC * _NS
_B_PER_W = _SEQ // _NW
_C = 32
_NCHUNK = _B_PER_W // _C

_mesh = plsc.VectorSubcoreMesh(core_axis_name="c", subcore_axis_name="s")


@functools.partial(
    pl.kernel,
    mesh=_mesh,
    out_type=jax.ShapeDtypeStruct((_SEQ, _D), jnp.float32),
    scratch_types=[
        pltpu.VMEM((_NCHUNK, _C), jnp.int32),
        pltpu.VMEM((_C, _D), jnp.float32),
        pltpu.SemaphoreType.DMA,
    ],
)
def _gather(table_hbm, idx_hbm, out_hbm, idx_v, buf, sem):
    wid = lax.axis_index("s") * _NC + lax.axis_index("c")
    base = wid * _B_PER_W
    pltpu.sync_copy(idx_hbm.at[wid], idx_v)
    for c in range(_NCHUNK):
        pltpu.async_copy(table_hbm.at[idx_v.at[c]], buf, sem).wait()
        pltpu.sync_copy(buf, out_hbm.at[pl.ds(base + c * _C, _C)])


def kernel(position_ids, PosEnc):
    table = PosEnc.reshape(_MAX_LEN, _D)
    idx = position_ids.astype(jnp.int32).reshape(_NW, _NCHUNK, _C)
    out = _gather(table, idx)
    return out.reshape(1, _SEQ, _D)


# speedup vs baseline: 1.0672x; 1.0672x over previous
import functools

import jax
import jax.numpy as jnp
from jax import lax
from jax.experimental import pallas as pl
from jax.experimental.pallas import tpu as pltpu
from jax.experimental.pallas import tpu_sc as plsc

_D = 1024
_MAX_LEN = 32768
_SEQ = 8192
_NC = 2
_NS = 16
_NW = _NC * _NS
_B_PER_W = _SEQ // _NW
_C = 32
_NCHUNK = _B_PER_W // _C

_mesh = plsc.VectorSubcoreMesh(core_axis_name="c", subcore_axis_name="s")


@functools.partial(
    pl.kernel,
    mesh=_mesh,
    out_type=jax.ShapeDtypeStruct((_SEQ, _D), jnp.float32),
    scratch_types=[
        pltpu.VMEM((_NCHUNK, _C), jnp.int32),
        pltpu.VMEM((_C, _D), jnp.float32),
        pltpu.VMEM((_C, _D), jnp.float32),
        pltpu.SemaphoreType.DMA,
        pltpu.SemaphoreType.DMA,
        pltpu.SemaphoreType.DMA,
        pltpu.SemaphoreType.DMA,
    ],
)
def _gather(table_hbm, idx_hbm, out_hbm, idx_v, buf0, buf1, g0, g1, s0, s1):
    wid = lax.axis_index("s") * _NC + lax.axis_index("c")
    base = wid * _B_PER_W
    bufs = (buf0, buf1)
    gsem = (g0, g1)
    ssem = (s0, s1)
    pltpu.sync_copy(idx_hbm.at[wid], idx_v)

    def start_gather(c):
        b = c & 1
        return pltpu.async_copy(table_hbm.at[idx_v.at[c]], bufs[b], gsem[b])

    def start_store(c):
        b = c & 1
        return pltpu.async_copy(
            bufs[b], out_hbm.at[pl.ds(base + c * _C, _C)], ssem[b]
        )

    gathers = [None] * _NCHUNK
    stores = [None] * _NCHUNK
    gathers[0] = start_gather(0)
    for c in range(_NCHUNK):
        gathers[c].wait()
        if c + 1 < _NCHUNK:
            if c >= 1:
                stores[c - 1].wait()
            gathers[c + 1] = start_gather(c + 1)
        stores[c] = start_store(c)
    stores[_NCHUNK - 2].wait()
    stores[_NCHUNK - 1].wait()


def kernel(position_ids, PosEnc):
    table = PosEnc.reshape(_MAX_LEN, _D)
    idx = position_ids.astype(jnp.int32).reshape(_NW, _NCHUNK, _C)
    out = _gather(table, idx)
    return out.reshape(1, _SEQ, _D)


# speedup vs baseline: 1.1443x; 1.0722x over previous
import functools

import jax
import jax.numpy as jnp
from jax import lax
from jax.experimental import pallas as pl
from jax.experimental.pallas import tpu as pltpu
from jax.experimental.pallas import tpu_sc as plsc

_D = 1024
_MAX_LEN = 32768
_SEQ = 8192
_NC = 2
_NS = 16
_NW = _NC * _NS
_B_PER_W = _SEQ // _NW
_C = 32
_NCHUNK = _B_PER_W // _C

_mesh = plsc.VectorSubcoreMesh(core_axis_name="c", subcore_axis_name="s")


_NBUF = 3


@functools.partial(
    pl.kernel,
    mesh=_mesh,
    out_type=jax.ShapeDtypeStruct((_SEQ, _D), jnp.float32),
    scratch_types=[
        pltpu.VMEM((_NCHUNK, _C), jnp.int32),
        pltpu.VMEM((_NBUF, _C, _D), jnp.float32),
        pltpu.SemaphoreType.DMA,
        pltpu.SemaphoreType.DMA,
        pltpu.SemaphoreType.DMA,
        pltpu.SemaphoreType.DMA,
        pltpu.SemaphoreType.DMA,
        pltpu.SemaphoreType.DMA,
    ],
)
def _gather(table_hbm, idx_hbm, out_hbm, idx_v, bufs, g0, g1, g2, s0, s1, s2):
    wid = lax.axis_index("s") * _NC + lax.axis_index("c")
    base = wid * _B_PER_W
    gsem = (g0, g1, g2)
    ssem = (s0, s1, s2)
    pltpu.sync_copy(idx_hbm.at[wid], idx_v)

    def start_gather(c):
        b = c % _NBUF
        return pltpu.async_copy(table_hbm.at[idx_v.at[c]], bufs.at[b], gsem[b])

    def start_store(c):
        b = c % _NBUF
        return pltpu.async_copy(
            bufs.at[b], out_hbm.at[pl.ds(base + c * _C, _C)], ssem[b]
        )

    gathers = [None] * _NCHUNK
    stores = [None] * _NCHUNK
    gathers[0] = start_gather(0)
    gathers[1] = start_gather(1)
    for c in range(_NCHUNK):
        if c + 2 < _NCHUNK:
            if c >= 1:
                stores[c - 1].wait()
            gathers[c + 2] = start_gather(c + 2)
        gathers[c].wait()
        stores[c] = start_store(c)
    stores[_NCHUNK - 3].wait()
    stores[_NCHUNK - 2].wait()
    stores[_NCHUNK - 1].wait()


def kernel(position_ids, PosEnc):
    table = PosEnc.reshape(_MAX_LEN, _D)
    idx = position_ids.astype(jnp.int32).reshape(_NW, _NCHUNK, _C)
    out = _gather(table, idx)
    return out.reshape(1, _SEQ, _D)
